# baseline (speedup 1.0000x reference)
import jax
import jax.numpy as jnp
from jax import lax
from jax.experimental import pallas as pl
from jax.experimental.pallas import tpu as pltpu
from jax.experimental.pallas import tpu_sc as plsc

N = 10000
E = 320000
F_IN = 128
F_OUT = 64
K = 8

NCORE = 2
NSUB = 16
FH = F_OUT // NCORE
EPT = E // NSUB
EROW = 160
EPAD = EROW * 128
NGRP = EROW // 4
NCMB = 4
RT = 640
RC = 160
NPAD = NSUB * RT
DUM = NPAD


def _mm_body(x_ref, w_ref, b_ref, c_ref):
    xv = x_ref[...]
    for k in range(K):
        for h in range(NCORE):
            acc = jnp.dot(xv, w_ref[k, :, h * FH:(h + 1) * FH],
                          preferred_element_type=jnp.float32)
            if k == 0:
                acc = acc + b_ref[h * FH:(h + 1) * FH][None, :]
            c_ref[k, h] = acc


def _sc_body(ei_ref, c_ref, out_ref, bA, bB, hbuf,
             spbuf, dpbuf, rb0, rb1, rb2, rb3, gbuf, bppbuf, cbuf, obuf, zrow,
             ones, dgbuf, ibuf,
             gacc, degsp,
             gs0, gs1, gs2, gs3, ss0, ss1, ss2, ss3):
    rbs = (rb0, rb1, rb2, rb3)
    gss = (gs0, gs1, gs2, gs3)
    sss = (ss0, ss1, ss2, ss3)
    cc = lax.axis_index("c")
    ss = lax.axis_index("s")
    r_lo = ss * RT

    def _zrow_init(r, carry):
        zrow[r, pl.ds(0, 16)] = jnp.zeros((16,), jnp.float32)
        zrow[r, pl.ds(16, 16)] = jnp.zeros((16,), jnp.float32)
        return carry
    lax.fori_loop(0, RC, _zrow_init, None)

    def _ones_init(i, carry):
        ones[pl.ds(i * 16, 16)] = jnp.ones((16,), jnp.float32)
        return carry
    lax.fori_loop(0, 8, _ones_init, None)

    def _dg_init(i, carry):
        dgbuf[pl.ds(i * 16, 16)] = jnp.zeros((16,), jnp.float32)
        return carry
    lax.fori_loop(0, RT // 16, _dg_init, None)

    pltpu.sync_copy(dgbuf, degsp.at[pl.ds(r_lo, RT)])

    def _zg(q, carry):
        pltpu.sync_copy(zrow, gacc.at[pl.ds(r_lo + q * RC, RC)])
        return carry
    lax.fori_loop(0, NCMB, _zg, None)

    pltpu.sync_copy(ei_ref.at[0, ss], spbuf)
    pltpu.sync_copy(ei_ref.at[1, ss], dpbuf)

    def _pp(t, carry):
        j = t // 8
        i = (t % 8) * 16
        sv = spbuf[j, pl.ds(i, 16)]
        dv = dpbuf[j, pl.ds(i, 16)]
        m = sv == dv
        spbuf[j, pl.ds(i, 16)] = jnp.where(m, jnp.int32(DUM), sv)
        dpbuf[j, pl.ds(i, 16)] = jnp.where(m, jnp.int32(DUM), dv)
        return carry
    lax.fori_loop(0, EROW * 8, _pp, None)

    plsc.subcore_barrier()

    def _deg(j, carry):
        pltpu.sync_copy(ones, degsp.at[spbuf.at[j]], add=True)
        return carry
    lax.fori_loop(0, EROW, _deg, None)

    plsc.subcore_barrier()

    pltpu.sync_copy(degsp.at[pl.ds(r_lo, RT)], dgbuf)

    def _isq(i, carry):
        d = dgbuf[pl.ds(i * 16, 16)]
        y = jnp.full((16,), 0.7, jnp.float32)
        for k in range(1, 10):
            y = y * jnp.where(d >= float(4 ** k), 0.5, 1.0)
        for _ in range(6):
            y = y * (1.5 - 0.5 * d * y * y)
        y = jnp.where(d > 0.5, y, 0.0)
        ibuf[pl.ds(i * 16, 16)] = y
        return carry
    lax.fori_loop(0, RT // 16, _isq, None)

    def _init(q, carry):
        r0 = r_lo + q * RC
        pltpu.sync_copy(c_ref.at[K - 1, cc, pl.ds(r0, RC)], cbuf)
        pltpu.sync_copy(cbuf, bA.at[cc, pl.ds(r0, RC)])

        def _rows(r, carry2):
            s = plsc.load_gather(ibuf, [jnp.full((16,), q * RC + r, jnp.int32)])
            for h in (0, 16):
                gbuf[r, pl.ds(h, 16)] = s * cbuf[r, pl.ds(h, 16)]
            return carry2
        lax.fori_loop(0, RC, _rows, None)
        pltpu.sync_copy(gbuf, hbuf.at[cc, pl.ds(r0, RC)])
        return carry
    lax.fori_loop(0, NCMB, _init, None)

    plsc.subcore_barrier()

    def _edge_pass():
        for u in range(4):
            pltpu.async_copy(hbuf.at[cc].at[spbuf.at[u]], rbs[u], gss[u])

        def _grp(j4, carry):
            base = 4 * j4
            for u in range(4):
                pltpu.make_async_copy(hbuf.at[cc].at[spbuf.at[0]], rbs[u], gss[u]).wait()
                pltpu.async_copy(rbs[u], gacc.at[dpbuf.at[base + u]], sss[u], add=True)

            @pl.when(j4 < NGRP - 1)
            def _prefetch():
                for u in range(4):
                    pltpu.make_async_copy(rbs[u], gacc.at[dpbuf.at[0]], sss[u]).wait()
                    pltpu.async_copy(hbuf.at[cc].at[spbuf.at[base + 4 + u]], rbs[u], gss[u])
            return carry
        lax.fori_loop(0, NGRP, _grp, None)
        for u in range(4):
            pltpu.make_async_copy(rbs[u], gacc.at[dpbuf.at[0]], sss[u]).wait()

    for k in range(K - 2, 0, -1):
        _edge_pass()
        plsc.subcore_barrier()
        slot = bB if k % 2 == 0 else bA
        first = (k == K - 2)

        def _cmb(q, carry, k=k, slot=slot, first=first):
            r0 = r_lo + q * RC
            pltpu.sync_copy(gacc.at[pl.ds(r0, RC)], gbuf)
            if not first:
                pltpu.sync_copy(slot.at[cc, pl.ds(r0, RC)], bppbuf)
            pltpu.sync_copy(c_ref.at[k, cc, pl.ds(r0, RC)], cbuf)

            def _rows(r, carry2):
                s = plsc.load_gather(ibuf, [jnp.full((16,), q * RC + r, jnp.int32)])
                for h in (0, 16):
                    v = cbuf[r, pl.ds(h, 16)] - (2.0 * s) * gbuf[r, pl.ds(h, 16)]
                    if not first:
                        v = v - bppbuf[r, pl.ds(h, 16)]
                    obuf[r, pl.ds(h, 16)] = v
                    gbuf[r, pl.ds(h, 16)] = s * v
                return carry2
            lax.fori_loop(0, RC, _rows, None)
            pltpu.sync_copy(obuf, slot.at[cc, pl.ds(r0, RC)])
            pltpu.sync_copy(gbuf, hbuf.at[cc, pl.ds(r0, RC)])
            pltpu.sync_copy(zrow, gacc.at[pl.ds(r0, RC)])
            return carry
        lax.fori_loop(0, NCMB, _cmb, None)
        plsc.subcore_barrier()

    _edge_pass()
    plsc.subcore_barrier()

    def _fin(q, carry):
        r0 = r_lo + q * RC
        pltpu.sync_copy(gacc.at[pl.ds(r0, RC)], gbuf)
        pltpu.sync_copy(bB.at[cc, pl.ds(r0, RC)], bppbuf)
        pltpu.sync_copy(c_ref.at[0, cc, pl.ds(r0, RC)], cbuf)

        def _rows(r, carry2):
            s = plsc.load_gather(ibuf, [jnp.full((16,), q * RC + r, jnp.int32)])
            for h in (0, 16):
                v = (cbuf[r, pl.ds(h, 16)] - s * gbuf[r, pl.ds(h, 16)]
                     - bppbuf[r, pl.ds(h, 16)])
                obuf[r, pl.ds(h, 16)] = jnp.maximum(v, 0.0)
            return carry2
        lax.fori_loop(0, RC, _rows, None)
        pltpu.sync_copy(obuf, out_ref.at[cc, pl.ds(r0, RC)])
        return carry
    lax.fori_loop(0, NCMB, _fin, None)


@jax.jit
def kernel(x, edge_index, W, b):
    c = pl.pallas_call(
        _mm_body,
        grid=(25,),
        in_specs=[
            pl.BlockSpec((N // 25, F_IN), lambda i: (i, 0)),
            pl.BlockSpec((K, F_IN, F_OUT), lambda i: (0, 0, 0)),
            pl.BlockSpec((F_OUT,), lambda i: (0,)),
        ],
        out_specs=pl.BlockSpec((K, NCORE, N // 25, FH), lambda i: (0, 0, i, 0)),
        out_shape=jax.ShapeDtypeStruct((K, NCORE, NPAD, FH), jnp.float32),
    )(x, W, b)

    ei3 = jnp.pad(
        edge_index.reshape(2, NSUB, EPT), ((0, 0), (0, 0), (0, EPAD - EPT))
    ).reshape(2, NSUB, EROW, 128)
    mesh = plsc.VectorSubcoreMesh(core_axis_name="c", subcore_axis_name="s")
    out2 = pl.kernel(
        _sc_body,
        out_type=(
            jax.ShapeDtypeStruct((NCORE, NPAD, FH), jnp.float32),
            jax.ShapeDtypeStruct((NCORE, NPAD, FH), jnp.float32),
            jax.ShapeDtypeStruct((NCORE, NPAD, FH), jnp.float32),
            jax.ShapeDtypeStruct((NCORE, NPAD + 8, FH), jnp.float32),
        ),
        mesh=mesh,
        compiler_params=pltpu.CompilerParams(
            use_tc_tiling_on_sc=False, needs_layout_passes=False),
        scratch_types=[
            pltpu.VMEM((EROW, 128), jnp.int32),
            pltpu.VMEM((EROW, 128), jnp.int32),
            pltpu.VMEM((128, FH), jnp.float32),
            pltpu.VMEM((128, FH), jnp.float32),
            pltpu.VMEM((128, FH), jnp.float32),
            pltpu.VMEM((128, FH), jnp.float32),
            pltpu.VMEM((RC, FH), jnp.float32),
            pltpu.VMEM((RC, FH), jnp.float32),
            pltpu.VMEM((RC, FH), jnp.float32),
            pltpu.VMEM((RC, FH), jnp.float32),
            pltpu.VMEM((RC, FH), jnp.float32),
            pltpu.VMEM((128,), jnp.float32),
            pltpu.VMEM((RT,), jnp.float32),
            pltpu.VMEM((RT,), jnp.float32),
            pltpu.VMEM_SHARED((NPAD + 8, FH), jnp.float32),
            pltpu.VMEM_SHARED((NPAD + 16,), jnp.float32),
        ] + [pltpu.SemaphoreType.DMA] * 8 + [
        ],
    )(ei3, c)[0]

    return out2[:, :N].transpose(1, 0, 2).reshape(N, F_OUT)


# speedup vs baseline: 1.4789x; 1.4789x over previous
import jax
import jax.numpy as jnp
from jax import lax
from jax.experimental import pallas as pl
from jax.experimental.pallas import tpu as pltpu
from jax.experimental.pallas import tpu_sc as plsc

N = 10000
E = 320000
F_IN = 128
F_OUT = 64
K = 8

NCORE = 2
NSUB = 16
FH = F_OUT // NCORE
EPT = E // NSUB
EROW = 160
EPAD = EROW * 128
DEPTH = 2
NGRP = EROW // DEPTH
NCMB = 4
RT = 640
RC = 160
NPAD = NSUB * RT
DUM = NPAD


def _mm_body(x_ref, w_ref, b_ref, c_ref):
    xv = x_ref[...]
    for k in range(K):
        for h in range(NCORE):
            acc = jnp.dot(xv, w_ref[k, :, h * FH:(h + 1) * FH],
                          preferred_element_type=jnp.float32)
            if k == 0:
                acc = acc + b_ref[h * FH:(h + 1) * FH][None, :]
            c_ref[k, h] = acc


def _sc_body(ei_ref, c_ref, out_ref, bA, bB,
             spbuf, dpbuf, rb0, rb1, rb2, rb3, gbuf, bppbuf, cbuf, obuf, zrow,
             ones, dgbuf, ibuf,
             hbuf, gacc, degsp,
             gs0, gs1, gs2, gs3, ss0, ss1, ss2, ss3):
    rbs = (rb0, rb1, rb2, rb3)
    gss = (gs0, gs1, gs2, gs3)
    sss = (ss0, ss1, ss2, ss3)
    cc = lax.axis_index("c")
    ss = lax.axis_index("s")
    r_lo = ss * RT

    def _zrow_init(r, carry):
        zrow[r, pl.ds(0, 16)] = jnp.zeros((16,), jnp.float32)
        zrow[r, pl.ds(16, 16)] = jnp.zeros((16,), jnp.float32)
        return carry
    lax.fori_loop(0, RC, _zrow_init, None)

    def _ones_init(i, carry):
        ones[pl.ds(i * 16, 16)] = jnp.ones((16,), jnp.float32)
        return carry
    lax.fori_loop(0, 8, _ones_init, None)

    def _dg_init(i, carry):
        dgbuf[pl.ds(i * 16, 16)] = jnp.zeros((16,), jnp.float32)
        return carry
    lax.fori_loop(0, RT // 16, _dg_init, None)

    pltpu.sync_copy(dgbuf, degsp.at[pl.ds(r_lo, RT)])

    def _zg(q, carry):
        pltpu.sync_copy(zrow, gacc.at[pl.ds(r_lo + q * RC, RC)])
        return carry
    lax.fori_loop(0, NCMB, _zg, None)

    pltpu.sync_copy(ei_ref.at[0, ss], spbuf)
    pltpu.sync_copy(ei_ref.at[1, ss], dpbuf)

    def _pp(t, carry):
        j = t // 8
        i = (t % 8) * 16
        sv = spbuf[j, pl.ds(i, 16)]
        dv = dpbuf[j, pl.ds(i, 16)]
        m = sv == dv
        spbuf[j, pl.ds(i, 16)] = jnp.where(m, jnp.int32(DUM), sv)
        dpbuf[j, pl.ds(i, 16)] = jnp.where(m, jnp.int32(DUM), dv)
        return carry
    lax.fori_loop(0, EROW * 8, _pp, None)

    plsc.subcore_barrier()

    def _deg(j, carry):
        pltpu.sync_copy(ones, degsp.at[spbuf.at[j]], add=True)
        return carry
    lax.fori_loop(0, EROW, _deg, None)

    plsc.subcore_barrier()

    pltpu.sync_copy(degsp.at[pl.ds(r_lo, RT)], dgbuf)

    def _isq(i, carry):
        d = dgbuf[pl.ds(i * 16, 16)]
        y = jnp.full((16,), 0.7, jnp.float32)
        for k in range(1, 10):
            y = y * jnp.where(d >= float(4 ** k), 0.5, 1.0)
        for _ in range(6):
            y = y * (1.5 - 0.5 * d * y * y)
        y = jnp.where(d > 0.5, y, 0.0)
        ibuf[pl.ds(i * 16, 16)] = y
        return carry
    lax.fori_loop(0, RT // 16, _isq, None)

    def _init(q, carry):
        r0 = r_lo + q * RC
        pltpu.sync_copy(c_ref.at[K - 1, cc, pl.ds(r0, RC)], cbuf)
        pltpu.sync_copy(cbuf, bA.at[cc, pl.ds(r0, RC)])

        def _rows(r, carry2):
            s = plsc.load_gather(ibuf, [jnp.full((16,), q * RC + r, jnp.int32)])
            for h in (0, 16):
                gbuf[r, pl.ds(h, 16)] = s * cbuf[r, pl.ds(h, 16)]
            return carry2
        lax.fori_loop(0, RC, _rows, None)
        pltpu.sync_copy(gbuf, hbuf.at[pl.ds(r0, RC)])
        return carry
    lax.fori_loop(0, NCMB, _init, None)

    plsc.subcore_barrier()

    def _edge_pass():
        for u in range(DEPTH):
            pltpu.async_copy(hbuf.at[spbuf.at[u]], rbs[u], gss[u])

        def _grp(j4, carry):
            base = DEPTH * j4
            for u in range(DEPTH):
                pltpu.make_async_copy(hbuf.at[spbuf.at[0]], rbs[u], gss[u]).wait()
                pltpu.async_copy(rbs[u], gacc.at[dpbuf.at[base + u]], sss[u], add=True)

            @pl.when(j4 < NGRP - 1)
            def _prefetch():
                for u in range(DEPTH):
                    pltpu.make_async_copy(rbs[u], gacc.at[dpbuf.at[0]], sss[u]).wait()
                    pltpu.async_copy(hbuf.at[spbuf.at[base + DEPTH + u]], rbs[u], gss[u])
            return carry
        lax.fori_loop(0, NGRP, _grp, None)
        for u in range(DEPTH):
            pltpu.make_async_copy(rbs[u], gacc.at[dpbuf.at[0]], sss[u]).wait()

    for k in range(K - 2, 0, -1):
        _edge_pass()
        plsc.subcore_barrier()
        slot = bB if k % 2 == 0 else bA
        first = (k == K - 2)

        def _cmb(q, carry, k=k, slot=slot, first=first):
            r0 = r_lo + q * RC
            pltpu.sync_copy(gacc.at[pl.ds(r0, RC)], gbuf)
            if not first:
                pltpu.sync_copy(slot.at[cc, pl.ds(r0, RC)], bppbuf)
            pltpu.sync_copy(c_ref.at[k, cc, pl.ds(r0, RC)], cbuf)

            def _rows(r, carry2):
                s = plsc.load_gather(ibuf, [jnp.full((16,), q * RC + r, jnp.int32)])
                for h in (0, 16):
                    v = cbuf[r, pl.ds(h, 16)] - (2.0 * s) * gbuf[r, pl.ds(h, 16)]
                    if not first:
                        v = v - bppbuf[r, pl.ds(h, 16)]
                    obuf[r, pl.ds(h, 16)] = v
                    gbuf[r, pl.ds(h, 16)] = s * v
                return carry2
            lax.fori_loop(0, RC, _rows, None)
            pltpu.sync_copy(obuf, slot.at[cc, pl.ds(r0, RC)])
            pltpu.sync_copy(gbuf, hbuf.at[pl.ds(r0, RC)])
            pltpu.sync_copy(zrow, gacc.at[pl.ds(r0, RC)])
            return carry
        lax.fori_loop(0, NCMB, _cmb, None)
        plsc.subcore_barrier()

    _edge_pass()
    plsc.subcore_barrier()

    def _fin(q, carry):
        r0 = r_lo + q * RC
        pltpu.sync_copy(gacc.at[pl.ds(r0, RC)], gbuf)
        pltpu.sync_copy(bB.at[cc, pl.ds(r0, RC)], bppbuf)
        pltpu.sync_copy(c_ref.at[0, cc, pl.ds(r0, RC)], cbuf)

        def _rows(r, carry2):
            s = plsc.load_gather(ibuf, [jnp.full((16,), q * RC + r, jnp.int32)])
            for h in (0, 16):
                v = (cbuf[r, pl.ds(h, 16)] - s * gbuf[r, pl.ds(h, 16)]
                     - bppbuf[r, pl.ds(h, 16)])
                obuf[r, pl.ds(h, 16)] = jnp.maximum(v, 0.0)
            return carry2
        lax.fori_loop(0, RC, _rows, None)
        pltpu.sync_copy(obuf, out_ref.at[cc, pl.ds(r0, RC)])
        return carry
    lax.fori_loop(0, NCMB, _fin, None)


@jax.jit
def kernel(x, edge_index, W, b):
    c = pl.pallas_call(
        _mm_body,
        grid=(25,),
        in_specs=[
            pl.BlockSpec((N // 25, F_IN), lambda i: (i, 0)),
            pl.BlockSpec((K, F_IN, F_OUT), lambda i: (0, 0, 0)),
            pl.BlockSpec((F_OUT,), lambda i: (0,)),
        ],
        out_specs=pl.BlockSpec((K, NCORE, N // 25, FH), lambda i: (0, 0, i, 0)),
        out_shape=jax.ShapeDtypeStruct((K, NCORE, NPAD, FH), jnp.float32),
    )(x, W, b)

    ei3 = jnp.pad(
        edge_index.reshape(2, NSUB, EPT), ((0, 0), (0, 0), (0, EPAD - EPT))
    ).reshape(2, NSUB, EROW, 128)
    mesh = plsc.VectorSubcoreMesh(core_axis_name="c", subcore_axis_name="s")
    out2 = pl.kernel(
        _sc_body,
        out_type=(
            jax.ShapeDtypeStruct((NCORE, NPAD, FH), jnp.float32),
            jax.ShapeDtypeStruct((NCORE, NPAD, FH), jnp.float32),
            jax.ShapeDtypeStruct((NCORE, NPAD, FH), jnp.float32),
        ),
        mesh=mesh,
        compiler_params=pltpu.CompilerParams(
            use_tc_tiling_on_sc=False, needs_layout_passes=False),
        scratch_types=[
            pltpu.VMEM((EROW, 128), jnp.int32),
            pltpu.VMEM((EROW, 128), jnp.int32),
            pltpu.VMEM((128, FH), jnp.float32),
            pltpu.VMEM((128, FH), jnp.float32),
            pltpu.VMEM((128, FH), jnp.float32),
            pltpu.VMEM((128, FH), jnp.float32),
            pltpu.VMEM((RC, FH), jnp.float32),
            pltpu.VMEM((RC, FH), jnp.float32),
            pltpu.VMEM((RC, FH), jnp.float32),
            pltpu.VMEM((RC, FH), jnp.float32),
            pltpu.VMEM((RC, FH), jnp.float32),
            pltpu.VMEM((128,), jnp.float32),
            pltpu.VMEM((RT,), jnp.float32),
            pltpu.VMEM((RT,), jnp.float32),
            pltpu.VMEM_SHARED((NPAD + 8, FH), jnp.float32),
            pltpu.VMEM_SHARED((NPAD + 8, FH), jnp.float32),
            pltpu.VMEM_SHARED((NPAD + 16,), jnp.float32),
        ] + [pltpu.SemaphoreType.DMA] * 8 + [
        ],
    )(ei3, c)[0]

    return out2[:, :N].transpose(1, 0, 2).reshape(N, F_OUT)


# speedup vs baseline: 1.5710x; 1.0623x over previous
import jax
import jax.numpy as jnp
from jax import lax
from jax.experimental import pallas as pl
from jax.experimental.pallas import tpu as pltpu
from jax.experimental.pallas import tpu_sc as plsc

N = 10000
E = 320000
F_IN = 128
F_OUT = 64
K = 8

NCORE = 2
NSUB = 16
FH = F_OUT // NCORE
EPT = E // NSUB
EROW = 160
EPAD = EROW * 128
DEPTH = 2
NGRP = EROW // DEPTH
NCMB = 4
RT = 640
RC = 160
NPAD = NSUB * RT
DUM = NPAD


def _mm_body(x_ref, w_ref, b_ref, c_ref):
    xv = x_ref[...]
    for k in range(K):
        for h in range(NCORE):
            acc = jnp.dot(xv, w_ref[k, :, h * FH:(h + 1) * FH],
                          preferred_element_type=jnp.float32)
            if k == 0:
                acc = acc + b_ref[h * FH:(h + 1) * FH][None, :]
            c_ref[k, h] = acc


def _sc_body(ei_ref, c_ref, out_ref, bA, bB,
             spbuf, dpbuf, rb0, rb1, rb2, rb3, gbuf, bppbuf, cbuf, obuf, zrow,
             ones, dgbuf, ibuf,
             hbuf, gacc, degsp,
             gs0, gs1, gs2, gs3, ss0, ss1, ss2, ss3):
    rbs = (rb0, rb1, rb2, rb3)
    gss = (gs0, gs1, gs2, gs3)
    sss = (ss0, ss1, ss2, ss3)
    cc = lax.axis_index("c")
    ss = lax.axis_index("s")
    r_lo = ss * RT

    def _zrow_init(r, carry):
        zrow[r, pl.ds(0, 16)] = jnp.zeros((16,), jnp.float32)
        zrow[r, pl.ds(16, 16)] = jnp.zeros((16,), jnp.float32)
        return carry
    lax.fori_loop(0, RC, _zrow_init, None)

    def _ones_init(i, carry):
        ones[pl.ds(i * 16, 16)] = jnp.ones((16,), jnp.float32)
        return carry
    lax.fori_loop(0, 8, _ones_init, None)

    def _dg_init(i, carry):
        dgbuf[pl.ds(i * 16, 16)] = jnp.zeros((16,), jnp.float32)
        return carry
    lax.fori_loop(0, RT // 16, _dg_init, None)

    pltpu.sync_copy(dgbuf, degsp.at[pl.ds(r_lo, RT)])

    def _zg(q, carry):
        pltpu.sync_copy(zrow, gacc.at[pl.ds(r_lo + q * RC, RC)])
        return carry
    lax.fori_loop(0, NCMB, _zg, None)

    pltpu.sync_copy(ei_ref.at[0, ss], spbuf)
    pltpu.sync_copy(ei_ref.at[1, ss], dpbuf)

    def _pp(t, carry):
        j = t // 8
        i = (t % 8) * 16
        sv = spbuf[j, pl.ds(i, 16)]
        dv = dpbuf[j, pl.ds(i, 16)]
        m = sv == dv
        spbuf[j, pl.ds(i, 16)] = jnp.where(m, jnp.int32(DUM), sv)
        dpbuf[j, pl.ds(i, 16)] = jnp.where(m, jnp.int32(DUM), dv)
        return carry
    lax.fori_loop(0, EROW * 8, _pp, None)

    plsc.subcore_barrier()

    dsems = gss + sss
    def _deg(g8, carry):
        for u in range(8):
            pltpu.async_copy(ones, degsp.at[spbuf.at[8 * g8 + u]], dsems[u], add=True)
        for u in range(8):
            pltpu.make_async_copy(ones, degsp.at[spbuf.at[0]], dsems[u]).wait()
        return carry
    lax.fori_loop(0, EROW // 8, _deg, None)

    plsc.subcore_barrier()

    pltpu.sync_copy(degsp.at[pl.ds(r_lo, RT)], dgbuf)

    def _isq(i, carry):
        d = dgbuf[pl.ds(i * 16, 16)]
        y = jnp.full((16,), 0.7, jnp.float32)
        for k in range(1, 10):
            y = y * jnp.where(d >= float(4 ** k), 0.5, 1.0)
        for _ in range(6):
            y = y * (1.5 - 0.5 * d * y * y)
        y = jnp.where(d > 0.5, y, 0.0)
        ibuf[pl.ds(i * 16, 16)] = y
        return carry
    lax.fori_loop(0, RT // 16, _isq, None)

    def _init(q, carry):
        r0 = r_lo + q * RC
        pltpu.sync_copy(c_ref.at[K - 1, cc, pl.ds(r0, RC)], cbuf)
        pltpu.sync_copy(cbuf, bA.at[cc, pl.ds(r0, RC)])

        def _rows(r, carry2):
            s = plsc.load_gather(ibuf, [jnp.full((16,), q * RC + r, jnp.int32)])
            for h in (0, 16):
                gbuf[r, pl.ds(h, 16)] = s * cbuf[r, pl.ds(h, 16)]
            return carry2
        lax.fori_loop(0, RC, _rows, None)
        pltpu.sync_copy(gbuf, hbuf.at[pl.ds(r0, RC)])
        return carry
    lax.fori_loop(0, NCMB, _init, None)

    plsc.subcore_barrier()

    def _edge_pass():
        for u in range(DEPTH):
            pltpu.async_copy(hbuf.at[spbuf.at[u]], rbs[u], gss[u])

        def _grp(j4, carry):
            base = DEPTH * j4
            for u in range(DEPTH):
                pltpu.make_async_copy(hbuf.at[spbuf.at[0]], rbs[u], gss[u]).wait()
                pltpu.async_copy(rbs[u], gacc.at[dpbuf.at[base + u]], sss[u], add=True)

            @pl.when(j4 < NGRP - 1)
            def _prefetch():
                for u in range(DEPTH):
                    pltpu.make_async_copy(rbs[u], gacc.at[dpbuf.at[0]], sss[u]).wait()
                    pltpu.async_copy(hbuf.at[spbuf.at[base + DEPTH + u]], rbs[u], gss[u])
            return carry
        lax.fori_loop(0, NGRP, _grp, None)
        for u in range(DEPTH):
            pltpu.make_async_copy(rbs[u], gacc.at[dpbuf.at[0]], sss[u]).wait()

    for k in range(K - 2, 0, -1):
        _edge_pass()
        plsc.subcore_barrier()
        slot = bB if k % 2 == 0 else bA
        first = (k == K - 2)

        def _cmb(q, carry, k=k, slot=slot, first=first):
            r0 = r_lo + q * RC

            @pl.when(q > 0)
            def _drain_prev():
                pltpu.make_async_copy(obuf, slot.at[cc, pl.ds(0, RC)], ss0).wait()
                pltpu.make_async_copy(gbuf, hbuf.at[pl.ds(0, RC)], ss1).wait()
                pltpu.make_async_copy(zrow, gacc.at[pl.ds(0, RC)], ss2).wait()

            pltpu.async_copy(gacc.at[pl.ds(r0, RC)], gbuf, gs0)
            if not first:
                pltpu.async_copy(slot.at[cc, pl.ds(r0, RC)], bppbuf, gs1)
            pltpu.async_copy(c_ref.at[k, cc, pl.ds(r0, RC)], cbuf, gs2)
            pltpu.make_async_copy(gacc.at[pl.ds(0, RC)], gbuf, gs0).wait()
            if not first:
                pltpu.make_async_copy(slot.at[cc, pl.ds(0, RC)], bppbuf, gs1).wait()
            pltpu.make_async_copy(c_ref.at[k, cc, pl.ds(0, RC)], cbuf, gs2).wait()

            def _rows(r, carry2):
                s = plsc.load_gather(ibuf, [jnp.full((16,), q * RC + r, jnp.int32)])
                for h in (0, 16):
                    v = cbuf[r, pl.ds(h, 16)] - (2.0 * s) * gbuf[r, pl.ds(h, 16)]
                    if not first:
                        v = v - bppbuf[r, pl.ds(h, 16)]
                    obuf[r, pl.ds(h, 16)] = v
                    gbuf[r, pl.ds(h, 16)] = s * v
                return carry2
            lax.fori_loop(0, RC, _rows, None)
            pltpu.async_copy(obuf, slot.at[cc, pl.ds(r0, RC)], ss0)
            pltpu.async_copy(gbuf, hbuf.at[pl.ds(r0, RC)], ss1)
            pltpu.async_copy(zrow, gacc.at[pl.ds(r0, RC)], ss2)
            return carry
        lax.fori_loop(0, NCMB, _cmb, None)
        pltpu.make_async_copy(obuf, slot.at[cc, pl.ds(0, RC)], ss0).wait()
        pltpu.make_async_copy(gbuf, hbuf.at[pl.ds(0, RC)], ss1).wait()
        pltpu.make_async_copy(zrow, gacc.at[pl.ds(0, RC)], ss2).wait()
        plsc.subcore_barrier()

    _edge_pass()
    plsc.subcore_barrier()

    def _fin(q, carry):
        r0 = r_lo + q * RC
        pltpu.sync_copy(gacc.at[pl.ds(r0, RC)], gbuf)
        pltpu.sync_copy(bB.at[cc, pl.ds(r0, RC)], bppbuf)
        pltpu.sync_copy(c_ref.at[0, cc, pl.ds(r0, RC)], cbuf)

        def _rows(r, carry2):
            s = plsc.load_gather(ibuf, [jnp.full((16,), q * RC + r, jnp.int32)])
            for h in (0, 16):
                v = (cbuf[r, pl.ds(h, 16)] - s * gbuf[r, pl.ds(h, 16)]
                     - bppbuf[r, pl.ds(h, 16)])
                obuf[r, pl.ds(h, 16)] = jnp.maximum(v, 0.0)
            return carry2
        lax.fori_loop(0, RC, _rows, None)
        pltpu.sync_copy(obuf, out_ref.at[pl.ds(r0, RC), pl.ds(FH * cc, FH)])
        return carry
    lax.fori_loop(0, NCMB, _fin, None)


@jax.jit
def kernel(x, edge_index, W, b):
    c = pl.pallas_call(
        _mm_body,
        grid=(25,),
        in_specs=[
            pl.BlockSpec((N // 25, F_IN), lambda i: (i, 0)),
            pl.BlockSpec((K, F_IN, F_OUT), lambda i: (0, 0, 0)),
            pl.BlockSpec((F_OUT,), lambda i: (0,)),
        ],
        out_specs=pl.BlockSpec((K, NCORE, N // 25, FH), lambda i: (0, 0, i, 0)),
        out_shape=jax.ShapeDtypeStruct((K, NCORE, NPAD, FH), jnp.float32),
    )(x, W, b)

    ei3 = jnp.pad(
        edge_index.reshape(2, NSUB, EPT), ((0, 0), (0, 0), (0, EPAD - EPT))
    ).reshape(2, NSUB, EROW, 128)
    mesh = plsc.VectorSubcoreMesh(core_axis_name="c", subcore_axis_name="s")
    out2 = pl.kernel(
        _sc_body,
        out_type=(
            jax.ShapeDtypeStruct((NPAD, F_OUT), jnp.float32),
            jax.ShapeDtypeStruct((NCORE, NPAD, FH), jnp.float32),
            jax.ShapeDtypeStruct((NCORE, NPAD, FH), jnp.float32),
        ),
        mesh=mesh,
        compiler_params=pltpu.CompilerParams(
            use_tc_tiling_on_sc=False, needs_layout_passes=False),
        scratch_types=[
            pltpu.VMEM((EROW, 128), jnp.int32),
            pltpu.VMEM((EROW, 128), jnp.int32),
            pltpu.VMEM((128, FH), jnp.float32),
            pltpu.VMEM((128, FH), jnp.float32),
            pltpu.VMEM((128, FH), jnp.float32),
            pltpu.VMEM((128, FH), jnp.float32),
            pltpu.VMEM((RC, FH), jnp.float32),
            pltpu.VMEM((RC, FH), jnp.float32),
            pltpu.VMEM((RC, FH), jnp.float32),
            pltpu.VMEM((RC, FH), jnp.float32),
            pltpu.VMEM((RC, FH), jnp.float32),
            pltpu.VMEM((128,), jnp.float32),
            pltpu.VMEM((RT,), jnp.float32),
            pltpu.VMEM((RT,), jnp.float32),
            pltpu.VMEM_SHARED((NPAD + 8, FH), jnp.float32),
            pltpu.VMEM_SHARED((NPAD + 8, FH), jnp.float32),
            pltpu.VMEM_SHARED((NPAD + 16,), jnp.float32),
        ] + [pltpu.SemaphoreType.DMA] * 8 + [
        ],
    )(ei3, c)[0]

    return out2[:N]
